# baseline (speedup 1.0000x reference)
import functools
import math

import jax
import jax.numpy as jnp
import numpy as np
from jax import lax
from jax.experimental import pallas as pl
from jax.experimental.pallas import tpu as pltpu
from jax.experimental.pallas import tpu_sc as plsc

D_MODEL = 1024
N_HEADS = 16
D_HEAD = 64
SEG = 128
S = 2048
SP = S + SEG
N_SEGS = SP // SEG
N_EXPERTS = 8
D_FFN = 512
EPS = 1e-4
RB = 128
NRB = S // RB

f32 = jnp.float32


def _gelu_exact(x):
    return 0.5 * x * (1.0 + lax.erf(x * (1.0 / math.sqrt(2.0))))


def _k1_body(x_ref, lnw_ref, lnb_ref, wx_ref, wy_ref, bx_ref, by_ref,
             a_ref, r_ref):
    x = x_ref[...]
    mu = jnp.mean(x, axis=1, keepdims=True)
    var = jnp.mean((x - mu) ** 2, axis=1, keepdims=True)
    a = (x - mu) * lax.rsqrt(var + 1e-5) * lnw_ref[...] + lnb_ref[...]
    a_ref[...] = a.astype(jnp.bfloat16)
    px = jnp.dot(a, wx_ref[...], preferred_element_type=f32) + bx_ref[...]
    py = jnp.dot(a, wy_ref[...], preferred_element_type=f32) + by_ref[...]
    r_ref[...] = px / (py + EPS)


def _k1(x2, lnw, lnb, wx, wy, bx, by):
    return pl.pallas_call(
        _k1_body,
        grid=(NRB,),
        in_specs=[
            pl.BlockSpec((RB, D_MODEL), lambda b: (b, 0)),
            pl.BlockSpec((1, D_MODEL), lambda b: (0, 0)),
            pl.BlockSpec((1, D_MODEL), lambda b: (0, 0)),
            pl.BlockSpec((D_MODEL, 128), lambda b: (0, 0)),
            pl.BlockSpec((D_MODEL, 128), lambda b: (0, 0)),
            pl.BlockSpec((1, 128), lambda b: (0, 0)),
            pl.BlockSpec((1, 128), lambda b: (0, 0)),
        ],
        out_specs=[
            pl.BlockSpec((RB, D_MODEL), lambda b: (b, 0)),
            pl.BlockSpec((RB, 128), lambda b: (b, 0)),
        ],
        out_shape=[
            jax.ShapeDtypeStruct((S, D_MODEL), jnp.bfloat16),
            jax.ShapeDtypeStruct((S, 128), f32),
        ],
    )(x2, lnw, lnb, wx, wy, bx, by)


def _k2_body(rt_ref, rc_ref, rr_ref, ri_ref):
    b = pl.program_id(0)
    h = pl.program_id(1)
    rall = rc_ref[...]
    lane = lax.broadcasted_iota(jnp.int32, (RB, 128), 1)
    c = jnp.sum(jnp.where(lane == h, rall, 0.0), axis=1, keepdims=True)
    rt = rt_ref[0]
    lt = (rt < c).astype(f32)
    le = (rt <= c).astype(f32)
    t_idx = lax.broadcasted_iota(jnp.int32, (RB, S), 1)
    s_idx = b * RB + lax.broadcasted_iota(jnp.int32, (RB, S), 0)
    before = jnp.where(t_idx < s_idx, le, lt)
    rank = jnp.sum(before, axis=1)
    rr_ref[0, 0, :] = rank
    ranki = rank.astype(jnp.int32)[:, None] * N_HEADS + h
    coli = jnp.where(lane == h, ranki, 0)

    @pl.when(h == 0)
    def _():
        ri_ref[...] = jnp.zeros_like(ri_ref)

    ri_ref[...] += coli


def _k2(rt3, rcp):
    return pl.pallas_call(
        _k2_body,
        grid=(NRB, N_HEADS),
        in_specs=[
            pl.BlockSpec((1, 1, S), lambda b, h: (h, 0, 0)),
            pl.BlockSpec((RB, 128), lambda b, h: (b, 0)),
        ],
        out_specs=[
            pl.BlockSpec((1, 1, RB), lambda b, h: (h, 0, b)),
            pl.BlockSpec((RB, 128), lambda b, h: (b, 0)),
        ],
        out_shape=[
            jax.ShapeDtypeStruct((N_HEADS, 1, S), f32),
            jax.ShapeDtypeStruct((S, 128), jnp.int32),
        ],
    )(rt3, rcp)


NROWS = S * N_HEADS
NW = 32
JCH = NROWS // NW // 128


def _sc_wid():
    return lax.axis_index("s") * 2 + lax.axis_index("c")


def _sc_gather_body(src_hbm, idx_hbm, out_hbm, idx_v, rows_v, sem):
    base = _sc_wid() * JCH
    pltpu.sync_copy(idx_hbm.at[pl.ds(base, JCH)], idx_v)
    cps = [pltpu.async_copy(src_hbm.at[idx_v.at[j]], rows_v.at[j], sem)
           for j in range(JCH)]
    for c in cps:
        c.wait()
    pltpu.sync_copy(rows_v, out_hbm.at[pl.ds(base, JCH)])


def _sc_gather(src2, idx2):
    k = functools.partial(
        pl.kernel,
        mesh=plsc.VectorSubcoreMesh(core_axis_name="c", subcore_axis_name="s"),
        compiler_params=pltpu.CompilerParams(use_tc_tiling_on_sc=False),
        out_type=jax.ShapeDtypeStruct((NROWS // 128, 128, D_HEAD), f32),
        scratch_types=[
            pltpu.VMEM((JCH, 128), jnp.int32),
            pltpu.VMEM((JCH, 128, D_HEAD), f32),
            pltpu.SemaphoreType.DMA,
        ],
    )(_sc_gather_body)
    return k(src2, idx2)


def _k3_body(rr_ref, a_ref, out_ref):
    b = pl.program_id(1)
    p0 = (b * RB) % S
    p_mat = p0 + lax.broadcasted_iota(jnp.int32, (RB, S), 0)
    lane = lax.broadcasted_iota(jnp.int32, (1, 128), 1)
    a = a_ref[...]
    acc = jnp.zeros((RB, 128), f32)
    for j in range(2):
        rrow = rr_ref[j, 0, :][None, :]
        g = (rrow == p_mat.astype(f32)).astype(jnp.bfloat16)
        mask = jnp.where((lane // D_HEAD) == j, 1.0, 0.0).astype(jnp.bfloat16)
        acc = acc + jnp.dot(g, a * mask, preferred_element_type=f32)
    out_ref[...] = acc


def _k3(ranksR, a):
    return pl.pallas_call(
        _k3_body,
        grid=(N_HEADS // 2, N_SEGS),
        in_specs=[
            pl.BlockSpec((2, 1, S), lambda hp, b: (hp, 0, 0)),
            pl.BlockSpec((S, 128), lambda hp, b: (0, hp)),
        ],
        out_specs=pl.BlockSpec((RB, 128), lambda hp, b: (b, hp)),
        out_shape=jax.ShapeDtypeStruct((SP, D_MODEL), f32),
    )(ranksR, a)


def _nt(x, w):
    return lax.dot_general(x, w, (((1,), (1,)), ((), ())),
                           preferred_element_type=f32)


def _k45_body(x_ref, wqk_ref, bqk_ref, wv_ref, bv_ref,
              wq_ref, bq_ref, wk_ref, bk_ref, wvi_ref, bvi_ref,
              wo_ref, bo_ref, out_ref):
    bf16 = jnp.bfloat16
    x = x_ref[...].astype(bf16)
    t1 = (_nt(x, wqk_ref[...]) + bqk_ref[...]).astype(bf16)
    t2 = (_nt(x, wv_ref[...]) + bv_ref[...]).astype(bf16)
    q = (_nt(t1, wq_ref[...]) + bq_ref[...]).astype(bf16)
    k = (_nt(t1, wk_ref[...]) + bk_ref[...]).astype(bf16)
    v = (_nt(t2, wvi_ref[...]) + bvi_ref[...]).astype(bf16)
    outs = []
    for h in range(N_HEADS):
        qh = q[:, h * D_HEAD:(h + 1) * D_HEAD]
        kh = k[:, h * D_HEAD:(h + 1) * D_HEAD]
        vh = v[:, h * D_HEAD:(h + 1) * D_HEAD]
        s = lax.dot_general(qh, kh, (((1,), (1,)), ((), ())),
                            preferred_element_type=f32) * (1.0 / 8.0)
        s = s - jnp.max(s, axis=1, keepdims=True)
        e = jnp.exp(s)
        p = (e / jnp.sum(e, axis=1, keepdims=True)).astype(bf16)
        outs.append(jnp.dot(p, vh, preferred_element_type=f32))
    att = jnp.concatenate(outs, axis=1).astype(bf16)
    out_ref[...] = _nt(att, wo_ref[...]) + bo_ref[...]


def _k45(xgp, wqk, bqk, wv, bv, in_w, in_b, wo, bo):
    wspec = pl.BlockSpec((D_MODEL, D_MODEL), lambda b: (0, 0))
    bspec = pl.BlockSpec((1, D_MODEL), lambda b: (0, 0))
    io = pl.BlockSpec((SEG, D_MODEL), lambda b: (b, 0))

    def inw(i):
        return pl.BlockSpec((D_MODEL, D_MODEL), lambda b: (i, 0))

    def inb(i):
        return pl.BlockSpec((1, D_MODEL), lambda b: (0, i))

    return pl.pallas_call(
        _k45_body,
        grid=(N_SEGS,),
        in_specs=[io, wspec, bspec, wspec, bspec,
                  inw(0), inb(0), inw(1), inb(1), inw(2), inb(2),
                  wspec, bspec],
        out_specs=io,
        out_shape=jax.ShapeDtypeStruct((SP, D_MODEL), f32),
    )(xgp, wqk, bqk, wv, bv, in_w, in_b, in_w, in_b, in_w, in_b, wo, bo)


def _k7_body(sc_ref, x1_ref, lnw_ref, lnb_ref, gw_ref, y1_ref, yn_ref, g_ref):
    x = x1_ref[...] + sc_ref[...]
    y1_ref[...] = x
    mu = jnp.mean(x, axis=1, keepdims=True)
    var = jnp.mean((x - mu) ** 2, axis=1, keepdims=True)
    yn = (x - mu) * lax.rsqrt(var + 1e-5) * lnw_ref[...] + lnb_ref[...]
    yn_ref[...] = yn
    logits = _nt(yn, gw_ref[...])
    lane = lax.broadcasted_iota(jnp.int32, (RB, 128), 1)
    logits = jnp.where(lane < N_EXPERTS, logits, -1e30)
    logits = logits - jnp.max(logits, axis=1, keepdims=True)
    e = jnp.exp(logits)
    g_ref[...] = e / jnp.sum(e, axis=1, keepdims=True)


def _k7(scat, x1, lnw, lnb, gwp):
    return pl.pallas_call(
        _k7_body,
        grid=(NRB,),
        in_specs=[
            pl.BlockSpec((RB, D_MODEL), lambda b: (b, 0)),
            pl.BlockSpec((RB, D_MODEL), lambda b: (b, 0)),
            pl.BlockSpec((1, D_MODEL), lambda b: (0, 0)),
            pl.BlockSpec((1, D_MODEL), lambda b: (0, 0)),
            pl.BlockSpec((128, D_MODEL), lambda b: (0, 0)),
        ],
        out_specs=[
            pl.BlockSpec((RB, D_MODEL), lambda b: (b, 0)),
            pl.BlockSpec((RB, D_MODEL), lambda b: (b, 0)),
            pl.BlockSpec((RB, 128), lambda b: (b, 0)),
        ],
        out_shape=[
            jax.ShapeDtypeStruct((S, D_MODEL), f32),
            jax.ShapeDtypeStruct((S, D_MODEL), f32),
            jax.ShapeDtypeStruct((S, 128), f32),
        ],
    )(scat, x1, lnw, lnb, gwp)


def _k8_body(yn_ref, g_ref, f1w_ref, f1b_ref, f2w_ref, f2b_ref, x2_ref,
             out_ref):
    e = pl.program_id(0)
    x = yn_ref[...].astype(jnp.bfloat16)
    h = _gelu_exact(_nt(x, f1w_ref[0]) + f1b_ref[0]).astype(jnp.bfloat16)
    eo = _nt(h, f2w_ref[0]) + f2b_ref[0]
    lane = lax.broadcasted_iota(jnp.int32, (S, 128), 1)
    ge = jnp.sum(jnp.where(lane == e, g_ref[...], 0.0), axis=1, keepdims=True)

    @pl.when(e == 0)
    def _():
        out_ref[...] = x2_ref[...]

    out_ref[...] += ge * eo


def _k8(yn, gates, f1w, f1b, f2w, f2b, x2):
    return pl.pallas_call(
        _k8_body,
        grid=(N_EXPERTS,),
        in_specs=[
            pl.BlockSpec((S, D_MODEL), lambda e: (0, 0)),
            pl.BlockSpec((S, 128), lambda e: (0, 0)),
            pl.BlockSpec((1, D_FFN, D_MODEL), lambda e: (e, 0, 0)),
            pl.BlockSpec((1, 1, D_FFN), lambda e: (e, 0, 0)),
            pl.BlockSpec((1, D_MODEL, D_FFN), lambda e: (e, 0, 0)),
            pl.BlockSpec((1, 1, D_MODEL), lambda e: (e, 0, 0)),
            pl.BlockSpec((S, D_MODEL), lambda e: (0, 0)),
        ],
        out_specs=pl.BlockSpec((S, D_MODEL), lambda e: (0, 0)),
        out_shape=jax.ShapeDtypeStruct((S, D_MODEL), f32),
    )(yn, gates, f1w, f1b, f2w, f2b, x2)


def _k9_body(a_ref, b_ref, o_ref):
    o_ref[...] = (a_ref[...] + b_ref[...]) * 0.5


def _k9(x1, x2):
    io = pl.BlockSpec((RB, D_MODEL), lambda b: (b, 0))
    return pl.pallas_call(
        _k9_body, grid=(NRB,), in_specs=[io, io], out_specs=io,
        out_shape=jax.ShapeDtypeStruct((S, D_MODEL), f32),
    )(x1, x2)


def _hash_mats(hash_w, hash_b):
    jj = np.arange(16)
    mask_x = np.zeros((16, 128), np.float32)
    mask_x[jj // 2, jj] = 1.0
    mask_y = np.zeros((16, 128), np.float32)
    mask_y[8 + jj // 2, jj] = 1.0
    hwx = hash_w[:8].transpose(2, 0, 1).reshape(D_HEAD, 16)
    hwy = hash_w[8:].transpose(2, 0, 1).reshape(D_HEAD, 16)
    hwx = jnp.pad(hwx, ((0, 0), (0, 112)))
    hwy = jnp.pad(hwy, ((0, 0), (0, 112)))
    wx = (jnp.asarray(mask_x)[:, None, :] * hwx[None, :, :]).reshape(D_MODEL, 128)
    wy = (jnp.asarray(mask_y)[:, None, :] * hwy[None, :, :]).reshape(D_MODEL, 128)
    return wx, wy, None


def _layer(x1, x2, lp):
    hash_w = lp['hash_w']
    hash_b = lp['hash_b']
    bflat = hash_b.reshape(32)
    bx = jnp.concatenate([bflat[:16], jnp.zeros((112,), f32)]).reshape(1, 128)
    by = jnp.concatenate([bflat[16:32], jnp.zeros((112,), f32)]).reshape(1, 128)
    wx, wy, _ = _hash_mats(hash_w, hash_b)

    a, r = _k1(x2, lp['ln_f_w'].reshape(1, -1), lp['ln_f_b'].reshape(1, -1),
               wx, wy, bx, by)

    rt3 = r[:, :N_HEADS].T.reshape(N_HEADS, 1, S)
    ranksR, ranksI = _k2(rt3, r)
    idx2 = ranksI[:, :N_HEADS].reshape(NROWS // 128, 128)
    xgp = _k3(ranksR, a)

    bf16 = jnp.bfloat16
    yseg = _k45(xgp,
                lp['proj_qk_w'].astype(bf16), lp['proj_qk_b'].reshape(1, -1),
                lp['proj_v_w'].astype(bf16), lp['proj_v_b'].reshape(1, -1),
                lp['in_proj_w'].astype(bf16), lp['in_proj_b'].reshape(1, -1),
                lp['out_proj_w'].astype(bf16), lp['out_proj_b'].reshape(1, -1))
    scat = _sc_gather(yseg.reshape(SP * N_HEADS, D_HEAD),
                      idx2).reshape(S, D_MODEL)

    gwp = jnp.zeros((128, D_MODEL), f32).at[:N_EXPERTS].set(lp['gate_w'])
    y1, yn, gates = _k7(scat, x1, lp['ln_g_w'].reshape(1, -1),
                        lp['ln_g_b'].reshape(1, -1), gwp)
    y2 = _k8(yn, gates, lp['fc1_w'].astype(bf16),
             lp['fc1_b'].reshape(N_EXPERTS, 1, D_FFN),
             lp['fc2_w'].astype(bf16),
             lp['fc2_b'].reshape(N_EXPERTS, 1, D_MODEL), x2)
    return y1, y2


def kernel(x, params):
    x0 = x[0]
    xr = jnp.repeat(x0, 2, axis=1)
    x1, x2 = xr[:, :D_MODEL], xr[:, D_MODEL:]
    for lp in params['layers']:
        x1, x2 = _layer(x1, x2, lp)
    out = _k9(x1, x2)
    return out[None]


# speedup vs baseline: 1.0509x; 1.0509x over previous
import functools
import math

import jax
import jax.numpy as jnp
import numpy as np
from jax import lax
from jax.experimental import pallas as pl
from jax.experimental.pallas import tpu as pltpu
from jax.experimental.pallas import tpu_sc as plsc

D_MODEL = 1024
N_HEADS = 16
D_HEAD = 64
SEG = 128
S = 2048
SP = S + SEG
N_SEGS = SP // SEG
N_EXPERTS = 8
D_FFN = 512
EPS = 1e-4
RB = 128
NRB = S // RB

f32 = jnp.float32


def _gelu_exact(x):
    return 0.5 * x * (1.0 + lax.erf(x * (1.0 / math.sqrt(2.0))))


def _k1_body(x_ref, lnw_ref, lnb_ref, wx_ref, wy_ref, bx_ref, by_ref,
             a_ref, r_ref):
    x = x_ref[...]
    mu = jnp.mean(x, axis=1, keepdims=True)
    var = jnp.mean((x - mu) ** 2, axis=1, keepdims=True)
    a = (x - mu) * lax.rsqrt(var + 1e-5) * lnw_ref[...] + lnb_ref[...]
    a_ref[...] = a
    px = jnp.dot(a, wx_ref[...], preferred_element_type=f32) + bx_ref[...]
    py = jnp.dot(a, wy_ref[...], preferred_element_type=f32) + by_ref[...]
    r_ref[...] = px / (py + EPS)


def _k1(x2, lnw, lnb, wx, wy, bx, by):
    return pl.pallas_call(
        _k1_body,
        grid=(NRB,),
        in_specs=[
            pl.BlockSpec((RB, D_MODEL), lambda b: (b, 0)),
            pl.BlockSpec((1, D_MODEL), lambda b: (0, 0)),
            pl.BlockSpec((1, D_MODEL), lambda b: (0, 0)),
            pl.BlockSpec((D_MODEL, 128), lambda b: (0, 0)),
            pl.BlockSpec((D_MODEL, 128), lambda b: (0, 0)),
            pl.BlockSpec((1, 128), lambda b: (0, 0)),
            pl.BlockSpec((1, 128), lambda b: (0, 0)),
        ],
        out_specs=[
            pl.BlockSpec((RB, D_MODEL), lambda b: (b, 0)),
            pl.BlockSpec((RB, 128), lambda b: (b, 0)),
        ],
        out_shape=[
            jax.ShapeDtypeStruct((S, D_MODEL), f32),
            jax.ShapeDtypeStruct((S, 128), f32),
        ],
    )(x2, lnw, lnb, wx, wy, bx, by)


def _k2_body(rt_ref, rc_ref, rr_ref, ri_ref):
    b = pl.program_id(0)
    h = pl.program_id(1)
    rall = rc_ref[...]
    lane = lax.broadcasted_iota(jnp.int32, (RB, 128), 1)
    c = jnp.sum(jnp.where(lane == h, rall, 0.0), axis=1, keepdims=True)
    rt = rt_ref[0]
    lt = (rt < c).astype(f32)
    le = (rt <= c).astype(f32)
    t_idx = lax.broadcasted_iota(jnp.int32, (RB, S), 1)
    s_idx = b * RB + lax.broadcasted_iota(jnp.int32, (RB, S), 0)
    before = jnp.where(t_idx < s_idx, le, lt)
    rank = jnp.sum(before, axis=1)
    rr_ref[0, 0, :] = rank
    ranki = rank.astype(jnp.int32)[:, None] * N_HEADS + h
    coli = jnp.where(lane == h, ranki, 0)

    @pl.when(h == 0)
    def _():
        ri_ref[...] = jnp.zeros_like(ri_ref)

    ri_ref[...] += coli


def _k2(rt3, rcp):
    return pl.pallas_call(
        _k2_body,
        grid=(NRB, N_HEADS),
        in_specs=[
            pl.BlockSpec((1, 1, S), lambda b, h: (h, 0, 0)),
            pl.BlockSpec((RB, 128), lambda b, h: (b, 0)),
        ],
        out_specs=[
            pl.BlockSpec((1, 1, RB), lambda b, h: (h, 0, b)),
            pl.BlockSpec((RB, 128), lambda b, h: (b, 0)),
        ],
        out_shape=[
            jax.ShapeDtypeStruct((N_HEADS, 1, S), f32),
            jax.ShapeDtypeStruct((S, 128), jnp.int32),
        ],
    )(rt3, rcp)


NROWS = S * N_HEADS
NW = 32
JCH = NROWS // NW // 128


def _sc_wid():
    return lax.axis_index("s") * 2 + lax.axis_index("c")


def _sc_gather_body(src_hbm, idx_hbm, out_hbm, idx_v, rows_v, sem):
    base = _sc_wid() * JCH
    pltpu.sync_copy(idx_hbm.at[pl.ds(base, JCH)], idx_v)
    cps = [pltpu.async_copy(src_hbm.at[idx_v.at[j]], rows_v.at[j], sem)
           for j in range(JCH)]
    for c in cps:
        c.wait()
    pltpu.sync_copy(rows_v, out_hbm.at[pl.ds(base, JCH)])


def _sc_gather(src2, idx2):
    k = functools.partial(
        pl.kernel,
        mesh=plsc.VectorSubcoreMesh(core_axis_name="c", subcore_axis_name="s"),
        compiler_params=pltpu.CompilerParams(use_tc_tiling_on_sc=False),
        out_type=jax.ShapeDtypeStruct((NROWS // 128, 128, D_HEAD), f32),
        scratch_types=[
            pltpu.VMEM((JCH, 128), jnp.int32),
            pltpu.VMEM((JCH, 128, D_HEAD), f32),
            pltpu.SemaphoreType.DMA,
        ],
    )(_sc_gather_body)
    return k(src2, idx2)


def _k3_body(rr_ref, a_ref, out_ref):
    b = pl.program_id(1)
    p0 = (b * RB) % S
    p_mat = p0 + lax.broadcasted_iota(jnp.int32, (RB, S), 0)
    lane = lax.broadcasted_iota(jnp.int32, (1, 128), 1)
    a = a_ref[...]
    acc = jnp.zeros((RB, 128), f32)
    for j in range(2):
        rrow = rr_ref[j, 0, :][None, :]
        g = (rrow == p_mat.astype(f32)).astype(f32)
        mask = jnp.where((lane // D_HEAD) == j, 1.0, 0.0)
        acc = acc + jnp.dot(g, a * mask, preferred_element_type=f32)
    out_ref[...] = acc


def _k3(ranksR, a):
    return pl.pallas_call(
        _k3_body,
        grid=(N_HEADS // 2, N_SEGS),
        in_specs=[
            pl.BlockSpec((2, 1, S), lambda hp, b: (hp, 0, 0)),
            pl.BlockSpec((S, 128), lambda hp, b: (0, hp)),
        ],
        out_specs=pl.BlockSpec((RB, 128), lambda hp, b: (b, hp)),
        out_shape=jax.ShapeDtypeStruct((SP, D_MODEL), f32),
    )(ranksR, a)


def _nt(x, w):
    return lax.dot_general(x, w, (((1,), (1,)), ((), ())),
                           preferred_element_type=f32)


def _k45_body(x_ref, wqk_ref, bqk_ref, wv_ref, bv_ref,
              wq_ref, bq_ref, wk_ref, bk_ref, wvi_ref, bvi_ref,
              wo_ref, bo_ref, out_ref):
    x = x_ref[...]
    t1 = _nt(x, wqk_ref[...]) + bqk_ref[...]
    t2 = _nt(x, wv_ref[...]) + bv_ref[...]
    q = _nt(t1, wq_ref[...]) + bq_ref[...]
    k = _nt(t1, wk_ref[...]) + bk_ref[...]
    v = _nt(t2, wvi_ref[...]) + bvi_ref[...]
    outs = []
    for h in range(N_HEADS):
        qh = q[:, h * D_HEAD:(h + 1) * D_HEAD]
        kh = k[:, h * D_HEAD:(h + 1) * D_HEAD]
        vh = v[:, h * D_HEAD:(h + 1) * D_HEAD]
        s = lax.dot_general(qh, kh, (((1,), (1,)), ((), ())),
                            preferred_element_type=f32) * (1.0 / 8.0)
        s = s - jnp.max(s, axis=1, keepdims=True)
        e = jnp.exp(s)
        p = e / jnp.sum(e, axis=1, keepdims=True)
        outs.append(jnp.dot(p, vh, preferred_element_type=f32))
    att = jnp.concatenate(outs, axis=1)
    out_ref[...] = _nt(att, wo_ref[...]) + bo_ref[...]


def _k45(xgp, wqk, bqk, wv, bv, in_w, in_b, wo, bo):
    wspec = pl.BlockSpec((D_MODEL, D_MODEL), lambda b: (0, 0))
    bspec = pl.BlockSpec((1, D_MODEL), lambda b: (0, 0))
    io = pl.BlockSpec((SEG, D_MODEL), lambda b: (b, 0))

    def inw(i):
        return pl.BlockSpec((D_MODEL, D_MODEL), lambda b: (i, 0))

    def inb(i):
        return pl.BlockSpec((1, D_MODEL), lambda b: (0, i))

    return pl.pallas_call(
        _k45_body,
        grid=(N_SEGS,),
        in_specs=[io, wspec, bspec, wspec, bspec,
                  inw(0), inb(0), inw(1), inb(1), inw(2), inb(2),
                  wspec, bspec],
        out_specs=io,
        out_shape=jax.ShapeDtypeStruct((SP, D_MODEL), f32),
    )(xgp, wqk, bqk, wv, bv, in_w, in_b, in_w, in_b, in_w, in_b, wo, bo)


def _k7_body(sc_ref, x1_ref, lnw_ref, lnb_ref, gw_ref, y1_ref, yn_ref, g_ref):
    x = x1_ref[...] + sc_ref[...]
    y1_ref[...] = x
    mu = jnp.mean(x, axis=1, keepdims=True)
    var = jnp.mean((x - mu) ** 2, axis=1, keepdims=True)
    yn = (x - mu) * lax.rsqrt(var + 1e-5) * lnw_ref[...] + lnb_ref[...]
    yn_ref[...] = yn
    logits = _nt(yn, gw_ref[...])
    lane = lax.broadcasted_iota(jnp.int32, (RB, 128), 1)
    logits = jnp.where(lane < N_EXPERTS, logits, -1e30)
    logits = logits - jnp.max(logits, axis=1, keepdims=True)
    e = jnp.exp(logits)
    g_ref[...] = e / jnp.sum(e, axis=1, keepdims=True)


def _k7(scat, x1, lnw, lnb, gwp):
    return pl.pallas_call(
        _k7_body,
        grid=(NRB,),
        in_specs=[
            pl.BlockSpec((RB, D_MODEL), lambda b: (b, 0)),
            pl.BlockSpec((RB, D_MODEL), lambda b: (b, 0)),
            pl.BlockSpec((1, D_MODEL), lambda b: (0, 0)),
            pl.BlockSpec((1, D_MODEL), lambda b: (0, 0)),
            pl.BlockSpec((128, D_MODEL), lambda b: (0, 0)),
        ],
        out_specs=[
            pl.BlockSpec((RB, D_MODEL), lambda b: (b, 0)),
            pl.BlockSpec((RB, D_MODEL), lambda b: (b, 0)),
            pl.BlockSpec((RB, 128), lambda b: (b, 0)),
        ],
        out_shape=[
            jax.ShapeDtypeStruct((S, D_MODEL), f32),
            jax.ShapeDtypeStruct((S, D_MODEL), f32),
            jax.ShapeDtypeStruct((S, 128), f32),
        ],
    )(scat, x1, lnw, lnb, gwp)


def _k8_body(yn_ref, g_ref, f1w_ref, f1b_ref, f2w_ref, f2b_ref, x2_ref,
             out_ref):
    e = pl.program_id(0)
    x = yn_ref[...]
    h = _gelu_exact(_nt(x, f1w_ref[0]) + f1b_ref[0])
    eo = _nt(h, f2w_ref[0]) + f2b_ref[0]
    lane = lax.broadcasted_iota(jnp.int32, (S, 128), 1)
    ge = jnp.sum(jnp.where(lane == e, g_ref[...], 0.0), axis=1, keepdims=True)

    @pl.when(e == 0)
    def _():
        out_ref[...] = x2_ref[...]

    out_ref[...] += ge * eo


def _k8(yn, gates, f1w, f1b, f2w, f2b, x2):
    return pl.pallas_call(
        _k8_body,
        grid=(N_EXPERTS,),
        in_specs=[
            pl.BlockSpec((S, D_MODEL), lambda e: (0, 0)),
            pl.BlockSpec((S, 128), lambda e: (0, 0)),
            pl.BlockSpec((1, D_FFN, D_MODEL), lambda e: (e, 0, 0)),
            pl.BlockSpec((1, 1, D_FFN), lambda e: (e, 0, 0)),
            pl.BlockSpec((1, D_MODEL, D_FFN), lambda e: (e, 0, 0)),
            pl.BlockSpec((1, 1, D_MODEL), lambda e: (e, 0, 0)),
            pl.BlockSpec((S, D_MODEL), lambda e: (0, 0)),
        ],
        out_specs=pl.BlockSpec((S, D_MODEL), lambda e: (0, 0)),
        out_shape=jax.ShapeDtypeStruct((S, D_MODEL), f32),
    )(yn, gates, f1w, f1b, f2w, f2b, x2)


def _k9_body(a_ref, b_ref, o_ref):
    o_ref[...] = (a_ref[...] + b_ref[...]) * 0.5


def _k9(x1, x2):
    io = pl.BlockSpec((RB, D_MODEL), lambda b: (b, 0))
    return pl.pallas_call(
        _k9_body, grid=(NRB,), in_specs=[io, io], out_specs=io,
        out_shape=jax.ShapeDtypeStruct((S, D_MODEL), f32),
    )(x1, x2)


def _hash_mats(hash_w, hash_b):
    jj = np.arange(16)
    mask_x = np.zeros((16, 128), np.float32)
    mask_x[jj // 2, jj] = 1.0
    mask_y = np.zeros((16, 128), np.float32)
    mask_y[8 + jj // 2, jj] = 1.0
    hwx = hash_w[:8].transpose(2, 0, 1).reshape(D_HEAD, 16)
    hwy = hash_w[8:].transpose(2, 0, 1).reshape(D_HEAD, 16)
    hwx = jnp.pad(hwx, ((0, 0), (0, 112)))
    hwy = jnp.pad(hwy, ((0, 0), (0, 112)))
    wx = (jnp.asarray(mask_x)[:, None, :] * hwx[None, :, :]).reshape(D_MODEL, 128)
    wy = (jnp.asarray(mask_y)[:, None, :] * hwy[None, :, :]).reshape(D_MODEL, 128)
    return wx, wy, None


def _layer(x1, x2, lp):
    hash_w = lp['hash_w']
    hash_b = lp['hash_b']
    bflat = hash_b.reshape(32)
    bx = jnp.concatenate([bflat[:16], jnp.zeros((112,), f32)]).reshape(1, 128)
    by = jnp.concatenate([bflat[16:32], jnp.zeros((112,), f32)]).reshape(1, 128)
    wx, wy, _ = _hash_mats(hash_w, hash_b)

    a, r = _k1(x2, lp['ln_f_w'].reshape(1, -1), lp['ln_f_b'].reshape(1, -1),
               wx, wy, bx, by)

    rt3 = r[:, :N_HEADS].T.reshape(N_HEADS, 1, S)
    ranksR, ranksI = _k2(rt3, r)
    idx2 = ranksI[:, :N_HEADS].reshape(NROWS // 128, 128)
    xgp = _k3(ranksR, a)

    yseg = _k45(xgp,
                lp['proj_qk_w'], lp['proj_qk_b'].reshape(1, -1),
                lp['proj_v_w'], lp['proj_v_b'].reshape(1, -1),
                lp['in_proj_w'], lp['in_proj_b'].reshape(1, -1),
                lp['out_proj_w'], lp['out_proj_b'].reshape(1, -1))
    scat = _sc_gather(yseg.reshape(SP * N_HEADS, D_HEAD),
                      idx2).reshape(S, D_MODEL)

    gwp = jnp.zeros((128, D_MODEL), f32).at[:N_EXPERTS].set(lp['gate_w'])
    y1, yn, gates = _k7(scat, x1, lp['ln_g_w'].reshape(1, -1),
                        lp['ln_g_b'].reshape(1, -1), gwp)
    y2 = _k8(yn, gates, lp['fc1_w'],
             lp['fc1_b'].reshape(N_EXPERTS, 1, D_FFN),
             lp['fc2_w'],
             lp['fc2_b'].reshape(N_EXPERTS, 1, D_MODEL), x2)
    return y1, y2


def kernel(x, params):
    x0 = x[0]
    xr = jnp.repeat(x0, 2, axis=1)
    x1, x2 = xr[:, :D_MODEL], xr[:, D_MODEL:]
    for lp in params['layers']:
        x1, x2 = _layer(x1, x2, lp)
    out = _k9(x1, x2)
    return out[None]
